# baseline (speedup 1.0000x reference)
import functools

import jax
import jax.numpy as jnp
from jax import lax
from jax.experimental import pallas as pl
from jax.experimental.pallas import tpu as pltpu
from jax.experimental.pallas import tpu_sc as plsc

NS = 16
L = 16


@functools.lru_cache(maxsize=None)
def _build_hist(n_idx):
    ic = n_idx // NS
    mesh = plsc.VectorSubcoreMesh(
        core_axis_name="c", subcore_axis_name="s", num_cores=1, num_subcores=NS
    )

    @functools.partial(
        pl.kernel,
        out_type=jax.ShapeDtypeStruct((1, 64), jnp.float32),
        mesh=mesh,
        scratch_types=[
            pltpu.VMEM((ic,), jnp.int32),
            pltpu.VMEM((64,), jnp.float32),
            pltpu.VMEM((NS * 64,), jnp.float32),
            pltpu.VMEM_SHARED((NS * 64,), jnp.float32),
        ],
        compiler_params=pltpu.CompilerParams(needs_layout_passes=False),
    )
    def hist(idx_hbm, out_hbm, idx_v, part_v, all_v, shared):
        sid = lax.axis_index("s")
        pltpu.sync_copy(idx_hbm.at[pl.ds(sid * ic, ic)], idx_v)

        zeros = jnp.zeros((L,), jnp.float32)
        ones = jnp.ones((L,), jnp.float32)
        for j in range(4):
            part_v[pl.ds(j * L, L)] = zeros

        def hbody(k, c):
            idx = idx_v[pl.ds(k * L, L)]
            plsc.addupdate_scatter(part_v, [idx], ones)
            return c

        lax.fori_loop(0, ic // L, hbody, 0)

        pltpu.sync_copy(part_v, shared.at[pl.ds(sid * 64, 64)])
        plsc.subcore_barrier()

        @pl.when(sid == 0)
        def _():
            pltpu.sync_copy(shared, all_v)
            cnt = [zeros] * 4
            for r in range(NS):
                for j in range(4):
                    cnt[j] = cnt[j] + all_v[pl.ds(r * 64 + j * L, L)]
            for j in range(4):
                part_v[pl.ds(j * L, L)] = cnt[j]
            pltpu.sync_copy(part_v, out_hbm.at[0])

    return hist


@functools.lru_cache(maxsize=None)
def _build_colsum(rows, cols, tiles=8):
    tile = rows // tiles

    def body(x_ref, o_ref, acc_ref):
        i = pl.program_id(0)

        @pl.when(i == 0)
        def _():
            acc_ref[...] = jnp.zeros_like(acc_ref)

        acc_ref[...] += jnp.sum(x_ref[...], axis=0, keepdims=True)

        @pl.when(i == tiles - 1)
        def _():
            o_ref[...] = acc_ref[...]

    return pl.pallas_call(
        body,
        grid=(tiles,),
        in_specs=[pl.BlockSpec((tile, cols), lambda i: (i, 0))],
        out_specs=pl.BlockSpec((1, cols), lambda i: (0, 0)),
        out_shape=jax.ShapeDtypeStruct((1, cols), jnp.float32),
        scratch_shapes=[pltpu.VMEM((1, cols), jnp.float32)],
    )


@functools.lru_cache(maxsize=None)
def _build_combine(cols, scale):
    def body(counts_ref, psum_ref, o_ref):
        total = jnp.sum(counts_ref[...] * psum_ref[...]) * scale
        o_ref[...] = jnp.broadcast_to(total, (1, 1))

    return pl.pallas_call(
        body,
        out_shape=jax.ShapeDtypeStruct((1, 1), jnp.float32),
    )


def kernel(router_probs, expert_indices, num_experts):
    B, M = router_probs.shape
    K = expert_indices.shape[1]
    assert M == 64, "kernel specialized for 64 experts"
    del num_experts
    idx_flat = expert_indices.reshape(-1).astype(jnp.int32)
    counts = _build_hist(B * K)(idx_flat)
    psum = _build_colsum(B, M)(router_probs)
    scale = float(M) / (float(B) * K * B)
    out = _build_combine(M, scale)(counts, psum)
    return out[0, 0]


# speedup vs baseline: 1.0326x; 1.0326x over previous
import functools

import jax
import jax.numpy as jnp
from jax import lax
from jax.experimental import pallas as pl
from jax.experimental.pallas import tpu as pltpu
from jax.experimental.pallas import tpu_sc as plsc

NS = 16
L = 16


@functools.lru_cache(maxsize=None)
def _build_hist(n_idx):
    ic = n_idx // NS
    mesh = plsc.VectorSubcoreMesh(
        core_axis_name="c", subcore_axis_name="s", num_cores=1, num_subcores=NS
    )

    @functools.partial(
        pl.kernel,
        out_type=jax.ShapeDtypeStruct((1, 64), jnp.float32),
        mesh=mesh,
        scratch_types=[
            pltpu.VMEM((ic,), jnp.int32),
            pltpu.VMEM((64,), jnp.float32),
            pltpu.VMEM((NS * 64,), jnp.float32),
            pltpu.VMEM_SHARED((NS * 64,), jnp.float32),
        ],
        compiler_params=pltpu.CompilerParams(needs_layout_passes=False),
    )
    def hist(idx_hbm, out_hbm, idx_v, part_v, all_v, shared):
        sid = lax.axis_index("s")
        pltpu.sync_copy(idx_hbm.at[pl.ds(sid * ic, ic)], idx_v)

        zeros = jnp.zeros((L,), jnp.float32)
        ones = jnp.ones((L,), jnp.float32)
        for j in range(4):
            part_v[pl.ds(j * L, L)] = zeros

        def hbody(k, c):
            idx = idx_v[pl.ds(k * L, L)]
            plsc.addupdate_scatter(part_v, [idx], ones)
            return c

        lax.fori_loop(0, ic // L, hbody, 0)

        pltpu.sync_copy(part_v, shared.at[pl.ds(sid * 64, 64)])
        plsc.subcore_barrier()

        @pl.when(sid == 0)
        def _():
            pltpu.sync_copy(shared, all_v)
            cnt = [zeros] * 4
            for r in range(NS):
                for j in range(4):
                    cnt[j] = cnt[j] + all_v[pl.ds(r * 64 + j * L, L)]
            for j in range(4):
                part_v[pl.ds(j * L, L)] = cnt[j]
            pltpu.sync_copy(part_v, out_hbm.at[0])

    return hist


@functools.lru_cache(maxsize=None)
def _build_colsum_dot(rows, cols, scale, tiles=8):
    tile = rows // tiles

    def body(counts_ref, x_ref, o_ref, acc_ref):
        i = pl.program_id(0)

        @pl.when(i == 0)
        def _():
            acc_ref[...] = jnp.zeros_like(acc_ref)

        acc_ref[...] += jnp.sum(x_ref[...], axis=0, keepdims=True)

        @pl.when(i == tiles - 1)
        def _():
            total = jnp.sum(acc_ref[...] * counts_ref[...]) * scale
            o_ref[...] = jnp.broadcast_to(total, (1, 1))

    return pl.pallas_call(
        body,
        grid=(tiles,),
        in_specs=[
            pl.BlockSpec((1, cols), lambda i: (0, 0)),
            pl.BlockSpec((tile, cols), lambda i: (i, 0)),
        ],
        out_specs=pl.BlockSpec((1, 1), lambda i: (0, 0)),
        out_shape=jax.ShapeDtypeStruct((1, 1), jnp.float32),
        scratch_shapes=[pltpu.VMEM((1, cols), jnp.float32)],
    )


def kernel(router_probs, expert_indices, num_experts):
    B, M = router_probs.shape
    K = expert_indices.shape[1]
    assert M == 64, "kernel specialized for 64 experts"
    del num_experts
    idx_flat = expert_indices.reshape(-1).astype(jnp.int32)
    counts = _build_hist(B * K)(idx_flat)
    scale = float(M) / (float(B) * K * B)
    out = _build_colsum_dot(B, M, scale)(counts, router_probs)
    return out[0, 0]
